# Initial kernel scaffold; baseline (speedup 1.0000x reference)
import jax
import jax.numpy as jnp
from jax.experimental import pallas as pl


def kernel(x, edge_index, edge_prob, W_proj, W_tp, a_src, a_trg, a_tp, W_skip, bias):
    raise NotImplementedError("write your pallas kernel here")


# speedup vs baseline: 39.5036x; 39.5036x over previous
import functools

import jax
import jax.numpy as jnp
from jax import lax
from jax.experimental import pallas as pl
from jax.experimental.pallas import tpu as pltpu
from jax.experimental.pallas import tpu_sc as plsc

N = 10000
E = 320000
D = 128
H = 8
F = 16
HF = H * F
ROW = 144
C = 80
NWORK = 32
EPT = E // NWORK
NCHUNK = EPT // C
RPT = 624
TAIL = N - 16 * RPT
ZR = 104
BN = 400
GRID = N // BN
EPB = E // D // GRID


def _tc1_body(x_ref, wp_ref, ws_ref, asrc_ref, atrg_ref, wtp_ref, atp_ref,
              sel_ref, ep_ref, proj_ref, tabs_ref, tabt_ref, skip_ref,
              cvec_ref, stats_ref):
    pid = pl.program_id(0)

    @pl.when(pid == 0)
    def _init():
        stats_ref[...] = jnp.full((8, 16), -jnp.inf, jnp.float32)
        stats_ref[2:3, :] = jnp.full((1, 16), jnp.max(ep_ref[...]), jnp.float32)

    xb = x_ref[...]
    pb = jnp.dot(xb, wp_ref[...], preferred_element_type=jnp.float32)
    proj_ref[...] = pb
    skip_ref[...] = jnp.dot(xb, ws_ref[...], preferred_element_type=jnp.float32)
    ts = jnp.dot(pb * asrc_ref[...], sel_ref[...],
                 preferred_element_type=jnp.float32)
    tt = jnp.dot(pb * atrg_ref[...], sel_ref[...],
                 preferred_element_type=jnp.float32)
    tabs_ref[...] = ts
    tabt_ref[...] = tt
    stats_ref[0:1, :] = jnp.maximum(stats_ref[0:1, :],
                                    jnp.max(ts, axis=0, keepdims=True))
    stats_ref[1:2, :] = jnp.maximum(stats_ref[1:2, :],
                                    jnp.max(tt, axis=0, keepdims=True))

    @pl.when(pid == pl.num_programs(0) - 1)
    def _finish():
        c16 = jnp.dot(wtp_ref[...] * atp_ref[...], sel_ref[...],
                      preferred_element_type=jnp.float32)
        bound = jnp.maximum(c16 * stats_ref[2:3, :], 0.0)
        raw = stats_ref[0:1, :] + stats_ref[1:2, :] + bound
        lk = jnp.where(raw > 0.0, raw, 0.2 * raw)
        mp = jnp.max(lk)
        iot = lax.broadcasted_iota(jnp.int32, (1, 16), 1)
        cvec_ref[...] = jnp.where(iot == 15, mp, c16)


def _tc2_body(part_ref, skip_ref, bias_ref, bsel_ref, out_ref):
    p = part_ref[0] + part_ref[1]
    num = p[:, 0:HF]
    den = jnp.dot(p[:, HF:ROW], bsel_ref[...],
                  preferred_element_type=jnp.float32)
    o = num / (den + 1e-16) + skip_ref[...] + bias_ref[...]
    out_ref[...] = jnp.where(o > 0.0, o, jnp.exp(o) - 1.0)


def _sc_edge(src, trg, ep, tabs, tabt, proj, cvec):
    mesh = plsc.VectorSubcoreMesh(core_axis_name="c", subcore_axis_name="s",
                                  num_cores=2, num_subcores=16)

    @functools.partial(
        pl.kernel,
        mesh=mesh,
        compiler_params=pltpu.CompilerParams(use_tc_tiling_on_sc=False),
        out_type=jax.ShapeDtypeStruct((2, N, ROW), jnp.float32),
        scratch_types=[
            pltpu.VMEM((C,), jnp.int32),
            pltpu.VMEM((C,), jnp.int32),
            pltpu.VMEM((C + 16,), jnp.float32),
            pltpu.VMEM((C, 16), jnp.float32),
            pltpu.VMEM((C, 16), jnp.float32),
            pltpu.VMEM((C, HF), jnp.float32),
            pltpu.VMEM((C, ROW), jnp.float32),
            pltpu.VMEM((16,), jnp.float32),
            pltpu.VMEM((ZR, ROW), jnp.float32),
            pltpu.VMEM_SHARED((N, ROW), jnp.float32),
            pltpu.SemaphoreType.DMA,
            pltpu.SemaphoreType.DMA,
        ],
    )
    def k(src_h, trg_h, ep_h, tabs_h, tabt_h, proj_h, cvec_h, out_h,
          src_v, trg_v, ep_v, srows, trows, prows, sbuf, cbuf, zbuf,
          accum, sem1, sem2):
        cid = lax.axis_index("c")
        sid = lax.axis_index("s")
        wid = sid * 2 + cid

        def zrow(r, carry):
            for j in range(ROW // 16):
                zbuf[r, pl.ds(16 * j, 16)] = jnp.zeros((16,), jnp.float32)
            return carry

        lax.fori_loop(0, ZR, zrow, 0)
        for z in range(RPT // ZR):
            pltpu.sync_copy(zbuf, accum.at[pl.ds(sid * RPT + z * ZR, ZR)])

        @pl.when(sid == 15)
        def _zero_tail():
            pltpu.sync_copy(zbuf.at[pl.ds(0, TAIL)],
                            accum.at[pl.ds(16 * RPT, TAIL)])
        pltpu.sync_copy(cvec_h, cbuf)
        plsc.subcore_barrier()

        cv = cbuf[...]
        mp = cv[15]
        lane_lt8 = lax.iota(jnp.int32, 16) < 8

        def chunk(g, carry):
            base = wid * EPT + g * C
            pltpu.sync_copy(src_h.at[pl.ds(base, C)], src_v)
            pltpu.sync_copy(trg_h.at[pl.ds(base, C)], trg_v)
            pltpu.sync_copy(ep_h.at[pl.ds(base, C)], ep_v.at[pl.ds(0, C)])
            cp1 = pltpu.async_copy(tabs_h.at[src_v], srows, sem1)
            cp2 = pltpu.async_copy(tabt_h.at[trg_v], trows, sem1)
            cp3 = pltpu.async_copy(proj_h.at[src_v], prows, sem2)
            cp1.wait()
            cp2.wait()
            cp3.wait()

            def edge(i, ecarry):
                ev = ep_v[pl.ds(i, 16)]
                s = srows[i, :] + trows[i, :] + ev[0] * cv
                s = jnp.where(s > 0.0, s, 0.2 * s)
                e = jnp.exp(s - mp)
                e = jnp.where(lane_lt8, e, 0.0)
                sbuf[i, pl.ds(HF, 16)] = e
                for j in range(H):
                    sbuf[i, pl.ds(16 * j, 16)] = prows[i, pl.ds(16 * j, 16)] * e[j]
                return ecarry

            lax.fori_loop(0, C, edge, 0)
            pltpu.sync_copy(sbuf, accum.at[trg_v], add=True)
            return carry

        lax.fori_loop(0, NCHUNK, chunk, 0)
        plsc.subcore_barrier()
        pltpu.sync_copy(accum.at[pl.ds(sid * RPT, RPT)],
                        out_h.at[cid, pl.ds(sid * RPT, RPT)])

        @pl.when(sid == 15)
        def _copy_tail():
            pltpu.sync_copy(accum.at[pl.ds(16 * RPT, TAIL)],
                            out_h.at[cid, pl.ds(16 * RPT, TAIL)])

    return k(src, trg, ep, tabs, tabt, proj, cvec)


def kernel(x, edge_index, edge_prob, W_proj, W_tp, a_src, a_trg, a_tp,
           W_skip, bias):
    src = edge_index[0]
    trg = edge_index[1]
    ep = edge_prob.reshape(E)
    wp_t = W_proj.T
    ws_t = W_skip.T
    asrc = a_src.reshape(1, HF)
    atrg = a_trg.reshape(1, HF)
    atp = a_tp.reshape(1, HF)
    wtp = W_tp.reshape(1, HF)
    sel16 = jnp.concatenate(
        [jnp.kron(jnp.eye(H, dtype=jnp.float32), jnp.ones((F, 1), jnp.float32)),
         jnp.zeros((HF, 8), jnp.float32)], axis=1)
    ep2d = ep.reshape(E // D, D)

    full = lambda shape: pl.BlockSpec(shape, lambda i: (0,) * len(shape))
    proj, tabs, tabt, skip, cvec = pl.pallas_call(
        _tc1_body,
        grid=(GRID,),
        in_specs=[
            pl.BlockSpec((BN, D), lambda i: (i, 0)),
            full((D, HF)), full((D, HF)),
            full((1, HF)), full((1, HF)), full((1, HF)), full((1, HF)),
            full((HF, 16)),
            full((E // D, D)),
        ],
        out_specs=[
            pl.BlockSpec((BN, HF), lambda i: (i, 0)),
            pl.BlockSpec((BN, 16), lambda i: (i, 0)),
            pl.BlockSpec((BN, 16), lambda i: (i, 0)),
            pl.BlockSpec((BN, HF), lambda i: (i, 0)),
            full((1, 16)),
        ],
        out_shape=[
            jax.ShapeDtypeStruct((N, HF), jnp.float32),
            jax.ShapeDtypeStruct((N, 16), jnp.float32),
            jax.ShapeDtypeStruct((N, 16), jnp.float32),
            jax.ShapeDtypeStruct((N, HF), jnp.float32),
            jax.ShapeDtypeStruct((1, 16), jnp.float32),
        ],
        scratch_shapes=[pltpu.VMEM((8, 16), jnp.float32)],
    )(x, wp_t, ws_t, asrc, atrg, wtp, atp, sel16, ep2d)

    partial = _sc_edge(src, trg, ep, tabs, tabt, proj, cvec.reshape(16))

    bias2 = bias.reshape(1, HF)
    bsel = jnp.concatenate(
        [jnp.kron(jnp.eye(H, dtype=jnp.float32), jnp.ones((1, F), jnp.float32)),
         jnp.zeros((8, HF), jnp.float32)], axis=0)
    out = pl.pallas_call(
        _tc2_body,
        grid=(GRID,),
        in_specs=[
            pl.BlockSpec((2, BN, ROW), lambda i: (0, i, 0)),
            pl.BlockSpec((BN, HF), lambda i: (i, 0)),
            full((1, HF)),
            full((16, HF)),
        ],
        out_specs=pl.BlockSpec((BN, HF), lambda i: (i, 0)),
        out_shape=jax.ShapeDtypeStruct((N, HF), jnp.float32),
    )(partial, skip, bias2, bsel)

    return out, edge_index, edge_prob


# speedup vs baseline: 49.4947x; 1.2529x over previous
import functools

import jax
import jax.numpy as jnp
from jax import lax
from jax.experimental import pallas as pl
from jax.experimental.pallas import tpu as pltpu
from jax.experimental.pallas import tpu_sc as plsc

N = 10000
E = 320000
D = 128
H = 8
F = 16
HF = H * F
ROW = 144
C = 80
NWORK = 32
EPT = E // NWORK
NCHUNK = EPT // C
RPT = 624
TAIL = N - 16 * RPT
ZR = 8
BN = 400
GRID = N // BN
EPB = E // D // GRID


def _tc1_body(x_ref, wp_ref, ws_ref, asrc_ref, atrg_ref, wtp_ref, atp_ref,
              sel_ref, ep_ref, proj_ref, tabs_ref, tabt_ref, skip_ref,
              cvec_ref, stats_ref):
    pid = pl.program_id(0)

    @pl.when(pid == 0)
    def _init():
        stats_ref[...] = jnp.full((8, 16), -jnp.inf, jnp.float32)
        stats_ref[2:3, :] = jnp.full((1, 16), jnp.max(ep_ref[...]), jnp.float32)

    xb = x_ref[...]
    pb = jnp.dot(xb, wp_ref[...], preferred_element_type=jnp.float32)
    proj_ref[...] = pb
    skip_ref[...] = jnp.dot(xb, ws_ref[...], preferred_element_type=jnp.float32)
    ts = jnp.dot(pb * asrc_ref[...], sel_ref[...],
                 preferred_element_type=jnp.float32)
    tt = jnp.dot(pb * atrg_ref[...], sel_ref[...],
                 preferred_element_type=jnp.float32)
    tabs_ref[...] = ts
    tabt_ref[...] = tt
    stats_ref[0:1, :] = jnp.maximum(stats_ref[0:1, :],
                                    jnp.max(ts, axis=0, keepdims=True))
    stats_ref[1:2, :] = jnp.maximum(stats_ref[1:2, :],
                                    jnp.max(tt, axis=0, keepdims=True))

    @pl.when(pid == pl.num_programs(0) - 1)
    def _finish():
        c16 = jnp.dot(wtp_ref[...] * atp_ref[...], sel_ref[...],
                      preferred_element_type=jnp.float32)
        bound = jnp.maximum(c16 * stats_ref[2:3, :], 0.0)
        raw = stats_ref[0:1, :] + stats_ref[1:2, :] + bound
        lk = jnp.where(raw > 0.0, raw, 0.2 * raw)
        mp = jnp.max(lk)
        iot = lax.broadcasted_iota(jnp.int32, (1, 16), 1)
        cvec_ref[...] = jnp.where(iot == 15, mp, c16)


def _tc2_body(part_ref, skip_ref, bias_ref, bsel_ref, out_ref):
    p = part_ref[0] + part_ref[1]
    num = p[:, 0:HF]
    den = jnp.dot(p[:, HF:ROW], bsel_ref[...],
                  preferred_element_type=jnp.float32)
    o = num / (den + 1e-16) + skip_ref[...] + bias_ref[...]
    out_ref[...] = jnp.where(o > 0.0, o, jnp.exp(o) - 1.0)


def _sc_edge(src, trg, ep, tabs, tabt, proj, cvec):
    mesh = plsc.VectorSubcoreMesh(core_axis_name="c", subcore_axis_name="s",
                                  num_cores=2, num_subcores=16)

    @functools.partial(
        pl.kernel,
        mesh=mesh,
        compiler_params=pltpu.CompilerParams(use_tc_tiling_on_sc=False),
        out_type=jax.ShapeDtypeStruct((2, N, ROW), jnp.float32),
        scratch_types=[
            [pltpu.VMEM((C,), jnp.int32)] * 2,
            [pltpu.VMEM((C,), jnp.int32)] * 2,
            [pltpu.VMEM((C + 16,), jnp.float32)] * 2,
            [pltpu.VMEM((C, 16), jnp.float32)] * 2,
            [pltpu.VMEM((C, 16), jnp.float32)] * 2,
            [pltpu.VMEM((C, HF), jnp.float32)] * 2,
            pltpu.VMEM((C, ROW), jnp.float32),
            pltpu.VMEM((16,), jnp.float32),
            pltpu.VMEM((ZR, ROW), jnp.float32),
            pltpu.VMEM_SHARED((N, ROW), jnp.float32),
            [pltpu.SemaphoreType.DMA] * 2,
            [pltpu.SemaphoreType.DMA] * 2,
        ],
    )
    def k(src_h, trg_h, ep_h, tabs_h, tabt_h, proj_h, cvec_h, out_h,
          src_v, trg_v, ep_v, srows, trows, prows, sbuf, cbuf, zbuf,
          accum, isem, gsem):
        cid = lax.axis_index("c")
        sid = lax.axis_index("s")
        wid = sid * 2 + cid

        def issue_idx(k_, p):
            base = wid * EPT + k_ * C
            pltpu.async_copy(src_h.at[pl.ds(base, C)], src_v[p], isem[p])
            pltpu.async_copy(trg_h.at[pl.ds(base, C)], trg_v[p], isem[p])
            pltpu.async_copy(ep_h.at[pl.ds(base, C)],
                             ep_v[p].at[pl.ds(0, C)], isem[p])

        def wait_idx(p):
            pltpu.make_async_copy(src_h.at[pl.ds(0, C)], src_v[p],
                                  isem[p]).wait()
            pltpu.make_async_copy(trg_h.at[pl.ds(0, C)], trg_v[p],
                                  isem[p]).wait()
            pltpu.make_async_copy(ep_h.at[pl.ds(0, C)],
                                  ep_v[p].at[pl.ds(0, C)], isem[p]).wait()

        def issue_gather(p):
            pltpu.async_copy(tabs_h.at[src_v[p]], srows[p], gsem[p])
            pltpu.async_copy(tabt_h.at[trg_v[p]], trows[p], gsem[p])
            pltpu.async_copy(proj_h.at[src_v[p]], prows[p], gsem[p])

        def wait_gather(p):
            pltpu.make_async_copy(tabs_h.at[pl.ds(0, C)], srows[p],
                                  gsem[p]).wait()
            pltpu.make_async_copy(tabt_h.at[pl.ds(0, C)], trows[p],
                                  gsem[p]).wait()
            pltpu.make_async_copy(proj_h.at[pl.ds(0, C)], prows[p],
                                  gsem[p]).wait()

        def zrow(r, carry):
            for j in range(ROW // 16):
                zbuf[r, pl.ds(16 * j, 16)] = jnp.zeros((16,), jnp.float32)
            return carry

        lax.fori_loop(0, ZR, zrow, 0)

        def zcopy(z, carry):
            pltpu.sync_copy(zbuf, accum.at[pl.ds(sid * RPT + z * ZR, ZR)])
            return carry

        lax.fori_loop(0, RPT // ZR, zcopy, 0)

        @pl.when(sid == 15)
        def _zero_tail():
            pltpu.sync_copy(zbuf.at[pl.ds(0, TAIL)],
                            accum.at[pl.ds(16 * RPT, TAIL)])
        pltpu.sync_copy(cvec_h, cbuf)
        plsc.subcore_barrier()

        cv = cbuf[...]
        mp = cv[15]
        lane_lt8 = lax.iota(jnp.int32, 16) < 8

        def compute_chunk(p):
            def edge(i, ecarry):
                ev = ep_v[p][pl.ds(i, 16)]
                s = srows[p][i, :] + trows[p][i, :] + ev[0] * cv
                s = jnp.where(s > 0.0, s, 0.2 * s)
                e = jnp.exp(s - mp)
                e = jnp.where(lane_lt8, e, 0.0)
                sbuf[i, pl.ds(HF, 16)] = e
                for j in range(H):
                    sbuf[i, pl.ds(16 * j, 16)] = (
                        prows[p][i, pl.ds(16 * j, 16)] * e[j])
                return ecarry

            lax.fori_loop(0, C, edge, 0)
            pltpu.sync_copy(sbuf, accum.at[trg_v[p]], add=True)

        issue_idx(0, 0)
        issue_idx(1, 1)
        wait_idx(0)
        issue_gather(0)

        def pair(gg, carry):
            k0 = 2 * gg
            wait_gather(0)
            wait_idx(1)
            issue_gather(1)
            compute_chunk(0)
            issue_idx(k0 + 2, 0)
            wait_gather(1)
            wait_idx(0)
            issue_gather(0)
            compute_chunk(1)

            @pl.when(gg < (NCHUNK - 1) // 2 - 1)
            def _more():
                issue_idx(k0 + 3, 1)

            return carry

        lax.fori_loop(0, (NCHUNK - 1) // 2, pair, 0)
        wait_gather(0)
        compute_chunk(0)
        plsc.subcore_barrier()
        pltpu.sync_copy(accum.at[pl.ds(sid * RPT, RPT)],
                        out_h.at[cid, pl.ds(sid * RPT, RPT)])

        @pl.when(sid == 15)
        def _copy_tail():
            pltpu.sync_copy(accum.at[pl.ds(16 * RPT, TAIL)],
                            out_h.at[cid, pl.ds(16 * RPT, TAIL)])

    return k(src, trg, ep, tabs, tabt, proj, cvec)


def kernel(x, edge_index, edge_prob, W_proj, W_tp, a_src, a_trg, a_tp,
           W_skip, bias):
    src = edge_index[0]
    trg = edge_index[1]
    ep = edge_prob.reshape(E)
    wp_t = W_proj.T
    ws_t = W_skip.T
    asrc = a_src.reshape(1, HF)
    atrg = a_trg.reshape(1, HF)
    atp = a_tp.reshape(1, HF)
    wtp = W_tp.reshape(1, HF)
    sel16 = jnp.concatenate(
        [jnp.kron(jnp.eye(H, dtype=jnp.float32), jnp.ones((F, 1), jnp.float32)),
         jnp.zeros((HF, 8), jnp.float32)], axis=1)
    ep2d = ep.reshape(E // D, D)

    full = lambda shape: pl.BlockSpec(shape, lambda i: (0,) * len(shape))
    proj, tabs, tabt, skip, cvec = pl.pallas_call(
        _tc1_body,
        grid=(GRID,),
        in_specs=[
            pl.BlockSpec((BN, D), lambda i: (i, 0)),
            full((D, HF)), full((D, HF)),
            full((1, HF)), full((1, HF)), full((1, HF)), full((1, HF)),
            full((HF, 16)),
            full((E // D, D)),
        ],
        out_specs=[
            pl.BlockSpec((BN, HF), lambda i: (i, 0)),
            pl.BlockSpec((BN, 16), lambda i: (i, 0)),
            pl.BlockSpec((BN, 16), lambda i: (i, 0)),
            pl.BlockSpec((BN, HF), lambda i: (i, 0)),
            full((1, 16)),
        ],
        out_shape=[
            jax.ShapeDtypeStruct((N, HF), jnp.float32),
            jax.ShapeDtypeStruct((N, 16), jnp.float32),
            jax.ShapeDtypeStruct((N, 16), jnp.float32),
            jax.ShapeDtypeStruct((N, HF), jnp.float32),
            jax.ShapeDtypeStruct((1, 16), jnp.float32),
        ],
        scratch_shapes=[pltpu.VMEM((8, 16), jnp.float32)],
    )(x, wp_t, ws_t, asrc, atrg, wtp, atp, sel16, ep2d)

    partial = _sc_edge(src, trg, ep, tabs, tabt, proj, cvec.reshape(16))

    bias2 = bias.reshape(1, HF)
    bsel = jnp.concatenate(
        [jnp.kron(jnp.eye(H, dtype=jnp.float32), jnp.ones((1, F), jnp.float32)),
         jnp.zeros((8, HF), jnp.float32)], axis=0)
    out = pl.pallas_call(
        _tc2_body,
        grid=(GRID,),
        in_specs=[
            pl.BlockSpec((2, BN, ROW), lambda i: (0, i, 0)),
            pl.BlockSpec((BN, HF), lambda i: (i, 0)),
            full((1, HF)),
            full((16, HF)),
        ],
        out_specs=pl.BlockSpec((BN, HF), lambda i: (i, 0)),
        out_shape=jax.ShapeDtypeStruct((N, HF), jnp.float32),
    )(partial, skip, bias2, bsel)

    return out, edge_index, edge_prob


# speedup vs baseline: 120.0475x; 2.4255x over previous
import functools

import jax
import jax.numpy as jnp
from jax import lax
from jax.experimental import pallas as pl
from jax.experimental.pallas import tpu as pltpu
from jax.experimental.pallas import tpu_sc as plsc

N = 10000
E = 320000
D = 128
H = 8
F = 16
HF = H * F
ROW = 144
C = 80
NWORK = 32
EPT = E // NWORK
NCHUNK = EPT // C
RPT = 624
TAIL = N - 16 * RPT
ZR = 8
BN = 400
GRID = N // BN
EPB = E // D // GRID


def _tc1_body(x_ref, wp_ref, ws_ref, asrc_ref, atrg_ref, wtp_ref, atp_ref,
              sel_ref, ep_ref, proj_ref, tabs_ref, tabt_ref, skip_ref,
              cvec_ref, stats_ref):
    pid = pl.program_id(0)

    @pl.when(pid == 0)
    def _init():
        stats_ref[...] = jnp.full((8, 16), -jnp.inf, jnp.float32)
        stats_ref[2:3, :] = jnp.full((1, 16), jnp.max(ep_ref[...]), jnp.float32)

    xb = x_ref[...]
    pb = jnp.dot(xb, wp_ref[...], preferred_element_type=jnp.float32)
    proj_ref[...] = pb
    skip_ref[...] = jnp.dot(xb, ws_ref[...], preferred_element_type=jnp.float32)
    ts = jnp.dot(pb * asrc_ref[...], sel_ref[...],
                 preferred_element_type=jnp.float32)
    tt = jnp.dot(pb * atrg_ref[...], sel_ref[...],
                 preferred_element_type=jnp.float32)
    tabs_ref[...] = ts
    tabt_ref[...] = tt
    stats_ref[0:1, :] = jnp.maximum(stats_ref[0:1, :],
                                    jnp.max(ts, axis=0, keepdims=True))
    stats_ref[1:2, :] = jnp.maximum(stats_ref[1:2, :],
                                    jnp.max(tt, axis=0, keepdims=True))

    @pl.when(pid == pl.num_programs(0) - 1)
    def _finish():
        c16 = jnp.dot(wtp_ref[...] * atp_ref[...], sel_ref[...],
                      preferred_element_type=jnp.float32)
        bound = jnp.maximum(c16 * stats_ref[2:3, :], 0.0)
        raw = stats_ref[0:1, :] + stats_ref[1:2, :] + bound
        lk = jnp.where(raw > 0.0, raw, 0.2 * raw)
        mp = jnp.max(lk)
        iot = lax.broadcasted_iota(jnp.int32, (1, 16), 1)
        cvec_ref[...] = jnp.where(iot == 15, mp, c16)


def _tc2_body(part_ref, skip_ref, bias_ref, bsel_ref, out_ref):
    p = part_ref[0] + part_ref[1]
    num = p[:, 0:HF]
    den = jnp.dot(p[:, HF:ROW], bsel_ref[...],
                  preferred_element_type=jnp.float32)
    o = num / (den + 1e-16) + skip_ref[...] + bias_ref[...]
    out_ref[...] = jnp.where(o > 0.0, o, jnp.exp(o) - 1.0)


def _sc_edge(src, trg, ep, tabs, tabt, proj, cvec):
    mesh = plsc.VectorSubcoreMesh(core_axis_name="c", subcore_axis_name="s",
                                  num_cores=2, num_subcores=16)

    @functools.partial(
        pl.kernel,
        mesh=mesh,
        compiler_params=pltpu.CompilerParams(use_tc_tiling_on_sc=False),
        out_type=jax.ShapeDtypeStruct((2, N, ROW), jnp.float32),
        scratch_types=[
            [pltpu.VMEM((C,), jnp.int32)] * 2,
            [pltpu.VMEM((C,), jnp.int32)] * 2,
            [pltpu.VMEM((C + 16,), jnp.float32)] * 2,
            [pltpu.VMEM((C, 16), jnp.float32)] * 2,
            [pltpu.VMEM((C, 16), jnp.float32)] * 2,
            [pltpu.VMEM((C, HF), jnp.float32)] * 2,
            pltpu.VMEM((C, ROW), jnp.float32),
            pltpu.VMEM((16,), jnp.float32),
            pltpu.VMEM((ZR, ROW), jnp.float32),
            pltpu.VMEM_SHARED((N, ROW), jnp.float32),
            [pltpu.SemaphoreType.DMA] * 2,
            [pltpu.SemaphoreType.DMA] * 2,
        ],
    )
    def k(src_h, trg_h, ep_h, tabs_h, tabt_h, proj_h, cvec_h, out_h,
          src_v, trg_v, ep_v, srows, trows, prows, sbuf, cbuf, zbuf,
          accum, isem, gsem):
        cid = lax.axis_index("c")
        sid = lax.axis_index("s")
        wid = sid * 2 + cid

        def issue_idx(k_, p):
            base = wid * EPT + k_ * C
            pltpu.async_copy(src_h.at[pl.ds(base, C)], src_v[p], isem[p])
            pltpu.async_copy(trg_h.at[pl.ds(base, C)], trg_v[p], isem[p])
            pltpu.async_copy(ep_h.at[pl.ds(base, C)],
                             ep_v[p].at[pl.ds(0, C)], isem[p])

        def wait_idx(p):
            pltpu.make_async_copy(src_h.at[pl.ds(0, C)], src_v[p],
                                  isem[p]).wait()
            pltpu.make_async_copy(trg_h.at[pl.ds(0, C)], trg_v[p],
                                  isem[p]).wait()
            pltpu.make_async_copy(ep_h.at[pl.ds(0, C)],
                                  ep_v[p].at[pl.ds(0, C)], isem[p]).wait()

        def issue_gather(p):
            pltpu.async_copy(tabs_h.at[src_v[p]], srows[p], gsem[p])
            pltpu.async_copy(tabt_h.at[trg_v[p]], trows[p], gsem[p])
            pltpu.async_copy(proj_h.at[src_v[p]], prows[p], gsem[p])

        def wait_gather(p):
            pltpu.make_async_copy(tabs_h.at[pl.ds(0, C)], srows[p],
                                  gsem[p]).wait()
            pltpu.make_async_copy(tabt_h.at[pl.ds(0, C)], trows[p],
                                  gsem[p]).wait()
            pltpu.make_async_copy(proj_h.at[pl.ds(0, C)], prows[p],
                                  gsem[p]).wait()

        def zrow(r, carry):
            for j in range(ROW // 16):
                zbuf[r, pl.ds(16 * j, 16)] = jnp.zeros((16,), jnp.float32)
            return carry

        lax.fori_loop(0, ZR, zrow, 0)

        def zcopy(z, carry):
            pltpu.sync_copy(zbuf, accum.at[pl.ds(sid * RPT + z * ZR, ZR)])
            return carry

        lax.fori_loop(0, RPT // ZR, zcopy, 0)

        @pl.when(sid == 15)
        def _zero_tail():
            pltpu.sync_copy(zbuf.at[pl.ds(0, TAIL)],
                            accum.at[pl.ds(16 * RPT, TAIL)])
        pltpu.sync_copy(cvec_h, cbuf)
        plsc.subcore_barrier()

        cv = cbuf[...]
        mp = cv[15]
        lane_lt8 = lax.iota(jnp.int32, 16) < 8

        def compute_chunk(p):
            @plsc.parallel_loop(0, C, unroll=4)
            def _edges(i):
                ev = ep_v[p][pl.ds(i, 16)]
                s = srows[p][i, :] + trows[p][i, :] + ev[0] * cv
                s = jnp.where(s > 0.0, s, 0.2 * s)
                e = jnp.exp(s - mp)
                e = jnp.where(lane_lt8, e, 0.0)
                sbuf[i, pl.ds(HF, 16)] = e
                for j in range(H):
                    sbuf[i, pl.ds(16 * j, 16)] = (
                        prows[p][i, pl.ds(16 * j, 16)] * e[j])

            pltpu.sync_copy(sbuf, accum.at[trg_v[p]], add=True)

        issue_idx(0, 0)
        issue_idx(1, 1)
        wait_idx(0)
        issue_gather(0)

        def pair(gg, carry):
            k0 = 2 * gg
            wait_gather(0)
            wait_idx(1)
            issue_gather(1)
            compute_chunk(0)
            issue_idx(k0 + 2, 0)
            wait_gather(1)
            wait_idx(0)
            issue_gather(0)
            compute_chunk(1)

            @pl.when(gg < (NCHUNK - 1) // 2 - 1)
            def _more():
                issue_idx(k0 + 3, 1)

            return carry

        lax.fori_loop(0, (NCHUNK - 1) // 2, pair, 0)
        wait_gather(0)
        compute_chunk(0)
        plsc.subcore_barrier()
        pltpu.sync_copy(accum.at[pl.ds(sid * RPT, RPT)],
                        out_h.at[cid, pl.ds(sid * RPT, RPT)])

        @pl.when(sid == 15)
        def _copy_tail():
            pltpu.sync_copy(accum.at[pl.ds(16 * RPT, TAIL)],
                            out_h.at[cid, pl.ds(16 * RPT, TAIL)])

    return k(src, trg, ep, tabs, tabt, proj, cvec)


def kernel(x, edge_index, edge_prob, W_proj, W_tp, a_src, a_trg, a_tp,
           W_skip, bias):
    src = edge_index[0]
    trg = edge_index[1]
    ep = edge_prob.reshape(E)
    wp_t = W_proj.T
    ws_t = W_skip.T
    asrc = a_src.reshape(1, HF)
    atrg = a_trg.reshape(1, HF)
    atp = a_tp.reshape(1, HF)
    wtp = W_tp.reshape(1, HF)
    sel16 = jnp.concatenate(
        [jnp.kron(jnp.eye(H, dtype=jnp.float32), jnp.ones((F, 1), jnp.float32)),
         jnp.zeros((HF, 8), jnp.float32)], axis=1)
    ep2d = ep.reshape(E // D, D)

    full = lambda shape: pl.BlockSpec(shape, lambda i: (0,) * len(shape))
    proj, tabs, tabt, skip, cvec = pl.pallas_call(
        _tc1_body,
        grid=(GRID,),
        in_specs=[
            pl.BlockSpec((BN, D), lambda i: (i, 0)),
            full((D, HF)), full((D, HF)),
            full((1, HF)), full((1, HF)), full((1, HF)), full((1, HF)),
            full((HF, 16)),
            full((E // D, D)),
        ],
        out_specs=[
            pl.BlockSpec((BN, HF), lambda i: (i, 0)),
            pl.BlockSpec((BN, 16), lambda i: (i, 0)),
            pl.BlockSpec((BN, 16), lambda i: (i, 0)),
            pl.BlockSpec((BN, HF), lambda i: (i, 0)),
            full((1, 16)),
        ],
        out_shape=[
            jax.ShapeDtypeStruct((N, HF), jnp.float32),
            jax.ShapeDtypeStruct((N, 16), jnp.float32),
            jax.ShapeDtypeStruct((N, 16), jnp.float32),
            jax.ShapeDtypeStruct((N, HF), jnp.float32),
            jax.ShapeDtypeStruct((1, 16), jnp.float32),
        ],
        scratch_shapes=[pltpu.VMEM((8, 16), jnp.float32)],
    )(x, wp_t, ws_t, asrc, atrg, wtp, atp, sel16, ep2d)

    partial = _sc_edge(src, trg, ep, tabs, tabt, proj, cvec.reshape(16))

    bias2 = bias.reshape(1, HF)
    bsel = jnp.concatenate(
        [jnp.kron(jnp.eye(H, dtype=jnp.float32), jnp.ones((1, F), jnp.float32)),
         jnp.zeros((8, HF), jnp.float32)], axis=0)
    out = pl.pallas_call(
        _tc2_body,
        grid=(GRID,),
        in_specs=[
            pl.BlockSpec((2, BN, ROW), lambda i: (0, i, 0)),
            pl.BlockSpec((BN, HF), lambda i: (i, 0)),
            full((1, HF)),
            full((16, HF)),
        ],
        out_specs=pl.BlockSpec((BN, HF), lambda i: (i, 0)),
        out_shape=jax.ShapeDtypeStruct((N, HF), jnp.float32),
    )(partial, skip, bias2, bsel)

    return out, edge_index, edge_prob


# speedup vs baseline: 120.2549x; 1.0017x over previous
import functools

import jax
import jax.numpy as jnp
from jax import lax
from jax.experimental import pallas as pl
from jax.experimental.pallas import tpu as pltpu
from jax.experimental.pallas import tpu_sc as plsc

N = 10000
E = 320000
D = 128
H = 8
F = 16
HF = H * F
ROW = 144
C = 80
NWORK = 32
EPT = E // NWORK
NCHUNK = EPT // C
RPT = 624
TAIL = N - 16 * RPT
ZR = 8
BN = 400
GRID = N // BN
EPB = E // D // GRID


def _tc1_body(x_ref, wp_ref, ws_ref, asrc_ref, atrg_ref, wtp_ref, atp_ref,
              sel_ref, ep_ref, proj_ref, tabs_ref, tabt_ref, skip_ref,
              cvec_ref, stats_ref):
    pid = pl.program_id(0)

    @pl.when(pid == 0)
    def _init():
        stats_ref[...] = jnp.full((8, 16), -jnp.inf, jnp.float32)
        stats_ref[2:3, :] = jnp.full((1, 16), jnp.max(ep_ref[...]), jnp.float32)

    xb = x_ref[...]
    pb = jnp.dot(xb, wp_ref[...], preferred_element_type=jnp.float32)
    proj_ref[...] = pb
    skip_ref[...] = jnp.dot(xb, ws_ref[...], preferred_element_type=jnp.float32)
    ts = jnp.dot(pb * asrc_ref[...], sel_ref[...],
                 preferred_element_type=jnp.float32)
    tt = jnp.dot(pb * atrg_ref[...], sel_ref[...],
                 preferred_element_type=jnp.float32)
    tabs_ref[...] = ts
    tabt_ref[...] = tt
    stats_ref[0:1, :] = jnp.maximum(stats_ref[0:1, :],
                                    jnp.max(ts, axis=0, keepdims=True))
    stats_ref[1:2, :] = jnp.maximum(stats_ref[1:2, :],
                                    jnp.max(tt, axis=0, keepdims=True))

    @pl.when(pid == pl.num_programs(0) - 1)
    def _finish():
        c16 = jnp.dot(wtp_ref[...] * atp_ref[...], sel_ref[...],
                      preferred_element_type=jnp.float32)
        bound = jnp.maximum(c16 * stats_ref[2:3, :], 0.0)
        raw = stats_ref[0:1, :] + stats_ref[1:2, :] + bound
        lk = jnp.where(raw > 0.0, raw, 0.2 * raw)
        mp = jnp.max(lk)
        iot = lax.broadcasted_iota(jnp.int32, (1, 16), 1)
        cvec_ref[...] = jnp.where(iot == 15, mp, c16)


def _tc2_body(part_ref, skip_ref, bias_ref, bsel_ref, out_ref):
    p = part_ref[0] + part_ref[1]
    num = p[:, 0:HF]
    den = jnp.dot(p[:, HF:ROW], bsel_ref[...],
                  preferred_element_type=jnp.float32)
    o = num / (den + 1e-16) + skip_ref[...] + bias_ref[...]
    out_ref[...] = jnp.where(o > 0.0, o, jnp.exp(o) - 1.0)


def _sc_edge(src, trg, ep, tabs, tabt, proj, cvec):
    mesh = plsc.VectorSubcoreMesh(core_axis_name="c", subcore_axis_name="s",
                                  num_cores=2, num_subcores=16)

    @functools.partial(
        pl.kernel,
        mesh=mesh,
        compiler_params=pltpu.CompilerParams(use_tc_tiling_on_sc=False),
        out_type=jax.ShapeDtypeStruct((2, N, ROW), jnp.float32),
        scratch_types=[
            [pltpu.VMEM((C,), jnp.int32)] * 2,
            [pltpu.VMEM((C,), jnp.int32)] * 2,
            [pltpu.VMEM((C + 16,), jnp.float32)] * 2,
            [pltpu.VMEM((C, 16), jnp.float32)] * 2,
            [pltpu.VMEM((C, 16), jnp.float32)] * 2,
            [pltpu.VMEM((C, HF), jnp.float32)] * 2,
            pltpu.VMEM((C, ROW), jnp.float32),
            pltpu.VMEM((16,), jnp.float32),
            pltpu.VMEM((ZR, ROW), jnp.float32),
            pltpu.VMEM_SHARED((N, ROW), jnp.float32),
            [pltpu.SemaphoreType.DMA] * 2,
            [pltpu.SemaphoreType.DMA] * 2,
        ],
    )
    def k(src_h, trg_h, ep_h, tabs_h, tabt_h, proj_h, cvec_h, out_h,
          src_v, trg_v, ep_v, srows, trows, prows, sbuf, cbuf, zbuf,
          accum, isem, gsem):
        cid = lax.axis_index("c")
        sid = lax.axis_index("s")
        wid = sid * 2 + cid

        def issue_idx(k_, p):
            base = wid * EPT + k_ * C
            pltpu.async_copy(src_h.at[pl.ds(base, C)], src_v[p], isem[p])
            pltpu.async_copy(trg_h.at[pl.ds(base, C)], trg_v[p], isem[p])
            pltpu.async_copy(ep_h.at[pl.ds(base, C)],
                             ep_v[p].at[pl.ds(0, C)], isem[p])

        def wait_idx(p):
            pltpu.make_async_copy(src_h.at[pl.ds(0, C)], src_v[p],
                                  isem[p]).wait()
            pltpu.make_async_copy(trg_h.at[pl.ds(0, C)], trg_v[p],
                                  isem[p]).wait()
            pltpu.make_async_copy(ep_h.at[pl.ds(0, C)],
                                  ep_v[p].at[pl.ds(0, C)], isem[p]).wait()

        def issue_gather(p):
            pltpu.async_copy(tabs_h.at[src_v[p]], srows[p], gsem[p])
            pltpu.async_copy(tabt_h.at[trg_v[p]], trows[p], gsem[p])
            pltpu.async_copy(proj_h.at[src_v[p]], prows[p], gsem[p])

        def wait_gather(p):
            pltpu.make_async_copy(tabs_h.at[pl.ds(0, C)], srows[p],
                                  gsem[p]).wait()
            pltpu.make_async_copy(tabt_h.at[pl.ds(0, C)], trows[p],
                                  gsem[p]).wait()
            pltpu.make_async_copy(proj_h.at[pl.ds(0, C)], prows[p],
                                  gsem[p]).wait()

        def zrow(r, carry):
            for j in range(ROW // 16):
                zbuf[r, pl.ds(16 * j, 16)] = jnp.zeros((16,), jnp.float32)
            return carry

        lax.fori_loop(0, ZR, zrow, 0)

        def zcopy(z, carry):
            pltpu.sync_copy(zbuf, accum.at[pl.ds(sid * RPT + z * ZR, ZR)])
            return carry

        lax.fori_loop(0, RPT // ZR, zcopy, 0)

        @pl.when(sid == 15)
        def _zero_tail():
            pltpu.sync_copy(zbuf.at[pl.ds(0, TAIL)],
                            accum.at[pl.ds(16 * RPT, TAIL)])
        pltpu.sync_copy(cvec_h, cbuf)
        plsc.subcore_barrier()

        cv = cbuf[...]
        mp = cv[15]
        lane_lt8 = lax.iota(jnp.int32, 16) < 8

        def compute_chunk(p):
            @plsc.parallel_loop(0, C, unroll=8)
            def _edges(i):
                ev = ep_v[p][pl.ds(i, 16)]
                s = srows[p][i, :] + trows[p][i, :] + ev[0] * cv
                s = jnp.where(s > 0.0, s, 0.2 * s)
                e = jnp.exp(s - mp)
                e = jnp.where(lane_lt8, e, 0.0)
                sbuf[i, pl.ds(HF, 16)] = e
                for j in range(H):
                    sbuf[i, pl.ds(16 * j, 16)] = (
                        prows[p][i, pl.ds(16 * j, 16)] * e[j])

            pltpu.sync_copy(sbuf, accum.at[trg_v[p]], add=True)

        issue_idx(0, 0)
        issue_idx(1, 1)
        wait_idx(0)
        issue_gather(0)

        def pair(gg, carry):
            k0 = 2 * gg
            wait_gather(0)
            wait_idx(1)
            issue_gather(1)
            compute_chunk(0)
            issue_idx(k0 + 2, 0)
            wait_gather(1)
            wait_idx(0)
            issue_gather(0)
            compute_chunk(1)

            @pl.when(gg < (NCHUNK - 1) // 2 - 1)
            def _more():
                issue_idx(k0 + 3, 1)

            return carry

        lax.fori_loop(0, (NCHUNK - 1) // 2, pair, 0)
        wait_gather(0)
        compute_chunk(0)
        plsc.subcore_barrier()
        pltpu.sync_copy(accum.at[pl.ds(sid * RPT, RPT)],
                        out_h.at[cid, pl.ds(sid * RPT, RPT)])

        @pl.when(sid == 15)
        def _copy_tail():
            pltpu.sync_copy(accum.at[pl.ds(16 * RPT, TAIL)],
                            out_h.at[cid, pl.ds(16 * RPT, TAIL)])

    return k(src, trg, ep, tabs, tabt, proj, cvec)


def kernel(x, edge_index, edge_prob, W_proj, W_tp, a_src, a_trg, a_tp,
           W_skip, bias):
    src = edge_index[0]
    trg = edge_index[1]
    ep = edge_prob.reshape(E)
    wp_t = W_proj.T
    ws_t = W_skip.T
    asrc = a_src.reshape(1, HF)
    atrg = a_trg.reshape(1, HF)
    atp = a_tp.reshape(1, HF)
    wtp = W_tp.reshape(1, HF)
    sel16 = jnp.concatenate(
        [jnp.kron(jnp.eye(H, dtype=jnp.float32), jnp.ones((F, 1), jnp.float32)),
         jnp.zeros((HF, 8), jnp.float32)], axis=1)
    ep2d = ep.reshape(E // D, D)

    full = lambda shape: pl.BlockSpec(shape, lambda i: (0,) * len(shape))
    proj, tabs, tabt, skip, cvec = pl.pallas_call(
        _tc1_body,
        grid=(GRID,),
        in_specs=[
            pl.BlockSpec((BN, D), lambda i: (i, 0)),
            full((D, HF)), full((D, HF)),
            full((1, HF)), full((1, HF)), full((1, HF)), full((1, HF)),
            full((HF, 16)),
            full((E // D, D)),
        ],
        out_specs=[
            pl.BlockSpec((BN, HF), lambda i: (i, 0)),
            pl.BlockSpec((BN, 16), lambda i: (i, 0)),
            pl.BlockSpec((BN, 16), lambda i: (i, 0)),
            pl.BlockSpec((BN, HF), lambda i: (i, 0)),
            full((1, 16)),
        ],
        out_shape=[
            jax.ShapeDtypeStruct((N, HF), jnp.float32),
            jax.ShapeDtypeStruct((N, 16), jnp.float32),
            jax.ShapeDtypeStruct((N, 16), jnp.float32),
            jax.ShapeDtypeStruct((N, HF), jnp.float32),
            jax.ShapeDtypeStruct((1, 16), jnp.float32),
        ],
        scratch_shapes=[pltpu.VMEM((8, 16), jnp.float32)],
    )(x, wp_t, ws_t, asrc, atrg, wtp, atp, sel16, ep2d)

    partial = _sc_edge(src, trg, ep, tabs, tabt, proj, cvec.reshape(16))

    bias2 = bias.reshape(1, HF)
    bsel = jnp.concatenate(
        [jnp.kron(jnp.eye(H, dtype=jnp.float32), jnp.ones((1, F), jnp.float32)),
         jnp.zeros((8, HF), jnp.float32)], axis=0)
    out = pl.pallas_call(
        _tc2_body,
        grid=(GRID,),
        in_specs=[
            pl.BlockSpec((2, BN, ROW), lambda i: (0, i, 0)),
            pl.BlockSpec((BN, HF), lambda i: (i, 0)),
            full((1, HF)),
            full((16, HF)),
        ],
        out_specs=pl.BlockSpec((BN, HF), lambda i: (i, 0)),
        out_shape=jax.ShapeDtypeStruct((N, HF), jnp.float32),
    )(partial, skip, bias2, bsel)

    return out, edge_index, edge_prob


# speedup vs baseline: 125.3985x; 1.0428x over previous
import functools

import jax
import jax.numpy as jnp
from jax import lax
from jax.experimental import pallas as pl
from jax.experimental.pallas import tpu as pltpu
from jax.experimental.pallas import tpu_sc as plsc

N = 10000
E = 320000
D = 128
H = 8
F = 16
HF = H * F
ROW = 144
C = 80
NWORK = 32
EPT = E // NWORK
NCHUNK = EPT // C
RPT = 624
TAIL = N - 16 * RPT
ZR = 8
BN = 400
GRID = N // BN
EPB = E // D // GRID


def _tc1_body(x_ref, wp_ref, asrc_ref, atrg_ref, wtp_ref, atp_ref,
              sel_ref, ep_ref, proj_ref, tabs_ref, tabt_ref,
              cvec_ref, stats_ref):
    pid = pl.program_id(0)

    @pl.when(pid == 0)
    def _init():
        stats_ref[...] = jnp.full((8, 16), -jnp.inf, jnp.float32)
        stats_ref[2:3, :] = jnp.full((1, 16), jnp.max(ep_ref[...]), jnp.float32)

    xb = x_ref[...]
    pb = jnp.dot(xb, wp_ref[...], preferred_element_type=jnp.float32)
    proj_ref[...] = pb
    ts = jnp.dot(pb * asrc_ref[...], sel_ref[...],
                 preferred_element_type=jnp.float32)
    tt = jnp.dot(pb * atrg_ref[...], sel_ref[...],
                 preferred_element_type=jnp.float32)
    tabs_ref[...] = ts
    tabt_ref[...] = tt
    stats_ref[0:1, :] = jnp.maximum(stats_ref[0:1, :],
                                    jnp.max(ts, axis=0, keepdims=True))
    stats_ref[1:2, :] = jnp.maximum(stats_ref[1:2, :],
                                    jnp.max(tt, axis=0, keepdims=True))

    @pl.when(pid == pl.num_programs(0) - 1)
    def _finish():
        c16 = jnp.dot(wtp_ref[...] * atp_ref[...], sel_ref[...],
                      preferred_element_type=jnp.float32)
        bound = jnp.maximum(c16 * stats_ref[2:3, :], 0.0)
        raw = stats_ref[0:1, :] + stats_ref[1:2, :] + bound
        lk = jnp.where(raw > 0.0, raw, 0.2 * raw)
        mp = jnp.max(lk)
        iot = lax.broadcasted_iota(jnp.int32, (1, 16), 1)
        cvec_ref[...] = jnp.where(iot == 15, mp, c16)


def _tc2_body(feat_ref, den_ref, x_ref, ws_ref, bias_ref, bsel_ref, out_ref):
    num = feat_ref[0] + feat_ref[1]
    den = jnp.dot(den_ref[0] + den_ref[1], bsel_ref[...],
                  preferred_element_type=jnp.float32)
    skip = jnp.dot(x_ref[...], ws_ref[...], preferred_element_type=jnp.float32)
    o = num / (den + 1e-16) + skip + bias_ref[...]
    out_ref[...] = jnp.where(o > 0.0, o, jnp.exp(o) - 1.0)


def _sc_edge(src, trg, ep, tabs, tabt, proj, cvec):
    mesh = plsc.VectorSubcoreMesh(core_axis_name="c", subcore_axis_name="s",
                                  num_cores=2, num_subcores=16)

    @functools.partial(
        pl.kernel,
        mesh=mesh,
        compiler_params=pltpu.CompilerParams(use_tc_tiling_on_sc=False),
        out_type=(jax.ShapeDtypeStruct((2, N, HF), jnp.float32),
                  jax.ShapeDtypeStruct((2, N, 16), jnp.float32)),
        scratch_types=[
            [pltpu.VMEM((C,), jnp.int32)] * 2,
            [pltpu.VMEM((C,), jnp.int32)] * 2,
            [pltpu.VMEM((C + 16,), jnp.float32)] * 2,
            [pltpu.VMEM((C, 16), jnp.float32)] * 2,
            [pltpu.VMEM((C, 16), jnp.float32)] * 2,
            [pltpu.VMEM((C, HF), jnp.float32)] * 2,
            pltpu.VMEM((C, ROW), jnp.float32),
            pltpu.VMEM((16,), jnp.float32),
            pltpu.VMEM((ZR, ROW), jnp.float32),
            pltpu.VMEM_SHARED((N, ROW), jnp.float32),
            [pltpu.SemaphoreType.DMA] * 2,
            [pltpu.SemaphoreType.DMA] * 2,
        ],
    )
    def k(src_h, trg_h, ep_h, tabs_h, tabt_h, proj_h, cvec_h,
          feat_h, den_h,
          src_v, trg_v, ep_v, srows, trows, prows, sbuf, cbuf, zbuf,
          accum, isem, gsem):
        cid = lax.axis_index("c")
        sid = lax.axis_index("s")
        wid = sid * 2 + cid

        def issue_idx(k_, p):
            base = wid * EPT + k_ * C
            pltpu.async_copy(src_h.at[pl.ds(base, C)], src_v[p], isem[p])
            pltpu.async_copy(trg_h.at[pl.ds(base, C)], trg_v[p], isem[p])
            pltpu.async_copy(ep_h.at[pl.ds(base, C)],
                             ep_v[p].at[pl.ds(0, C)], isem[p])

        def wait_idx(p):
            pltpu.make_async_copy(src_h.at[pl.ds(0, C)], src_v[p],
                                  isem[p]).wait()
            pltpu.make_async_copy(trg_h.at[pl.ds(0, C)], trg_v[p],
                                  isem[p]).wait()
            pltpu.make_async_copy(ep_h.at[pl.ds(0, C)],
                                  ep_v[p].at[pl.ds(0, C)], isem[p]).wait()

        def issue_gather(p):
            pltpu.async_copy(tabs_h.at[src_v[p]], srows[p], gsem[p])
            pltpu.async_copy(tabt_h.at[trg_v[p]], trows[p], gsem[p])
            pltpu.async_copy(proj_h.at[src_v[p]], prows[p], gsem[p])

        def wait_gather(p):
            pltpu.make_async_copy(tabs_h.at[pl.ds(0, C)], srows[p],
                                  gsem[p]).wait()
            pltpu.make_async_copy(tabt_h.at[pl.ds(0, C)], trows[p],
                                  gsem[p]).wait()
            pltpu.make_async_copy(proj_h.at[pl.ds(0, C)], prows[p],
                                  gsem[p]).wait()

        def zrow(r, carry):
            for j in range(ROW // 16):
                zbuf[r, pl.ds(16 * j, 16)] = jnp.zeros((16,), jnp.float32)
            return carry

        lax.fori_loop(0, ZR, zrow, 0)

        def zcopy(z, carry):
            pltpu.sync_copy(zbuf, accum.at[pl.ds(sid * RPT + z * ZR, ZR)])
            return carry

        lax.fori_loop(0, RPT // ZR, zcopy, 0)

        @pl.when(sid == 15)
        def _zero_tail():
            pltpu.sync_copy(zbuf.at[pl.ds(0, TAIL)],
                            accum.at[pl.ds(16 * RPT, TAIL)])
        pltpu.sync_copy(cvec_h, cbuf)
        plsc.subcore_barrier()

        cv = cbuf[...]
        mp = cv[15]
        lane_lt8 = lax.iota(jnp.int32, 16) < 8

        def compute_chunk(p):
            @plsc.parallel_loop(0, C, unroll=8)
            def _edges(i):
                ev = ep_v[p][pl.ds(i, 16)]
                s = srows[p][i, :] + trows[p][i, :] + ev[0] * cv
                s = jnp.where(s > 0.0, s, 0.2 * s)
                e = jnp.exp(s - mp)
                e = jnp.where(lane_lt8, e, 0.0)
                sbuf[i, pl.ds(HF, 16)] = e
                for j in range(H):
                    sbuf[i, pl.ds(16 * j, 16)] = (
                        prows[p][i, pl.ds(16 * j, 16)] * e[j])

            pltpu.sync_copy(sbuf, accum.at[trg_v[p]], add=True)

        issue_idx(0, 0)
        issue_idx(1, 1)
        wait_idx(0)
        issue_gather(0)

        def pair(gg, carry):
            k0 = 2 * gg
            wait_gather(0)
            wait_idx(1)
            issue_gather(1)
            compute_chunk(0)
            issue_idx(k0 + 2, 0)
            wait_gather(1)
            wait_idx(0)
            issue_gather(0)
            compute_chunk(1)

            @pl.when(gg < (NCHUNK - 1) // 2 - 1)
            def _more():
                issue_idx(k0 + 3, 1)

            return carry

        lax.fori_loop(0, (NCHUNK - 1) // 2, pair, 0)
        wait_gather(0)
        compute_chunk(0)
        plsc.subcore_barrier()
        pltpu.sync_copy(accum.at[pl.ds(sid * RPT, RPT), pl.ds(0, HF)],
                        feat_h.at[cid, pl.ds(sid * RPT, RPT)])
        pltpu.sync_copy(accum.at[pl.ds(sid * RPT, RPT), pl.ds(HF, 16)],
                        den_h.at[cid, pl.ds(sid * RPT, RPT)])

        @pl.when(sid == 15)
        def _copy_tail():
            pltpu.sync_copy(accum.at[pl.ds(16 * RPT, TAIL), pl.ds(0, HF)],
                            feat_h.at[cid, pl.ds(16 * RPT, TAIL)])
            pltpu.sync_copy(accum.at[pl.ds(16 * RPT, TAIL), pl.ds(HF, 16)],
                            den_h.at[cid, pl.ds(16 * RPT, TAIL)])

    return k(src, trg, ep, tabs, tabt, proj, cvec)


def kernel(x, edge_index, edge_prob, W_proj, W_tp, a_src, a_trg, a_tp,
           W_skip, bias):
    src = edge_index[0]
    trg = edge_index[1]
    ep = edge_prob.reshape(E)
    wp_t = W_proj.T
    ws_t = W_skip.T
    asrc = a_src.reshape(1, HF)
    atrg = a_trg.reshape(1, HF)
    atp = a_tp.reshape(1, HF)
    wtp = W_tp.reshape(1, HF)
    sel16 = jnp.concatenate(
        [jnp.kron(jnp.eye(H, dtype=jnp.float32), jnp.ones((F, 1), jnp.float32)),
         jnp.zeros((HF, 8), jnp.float32)], axis=1)
    ep2d = ep.reshape(E // D, D)

    full = lambda shape: pl.BlockSpec(shape, lambda i: (0,) * len(shape))
    proj, tabs, tabt, cvec = pl.pallas_call(
        _tc1_body,
        grid=(GRID,),
        in_specs=[
            pl.BlockSpec((BN, D), lambda i: (i, 0)),
            full((D, HF)),
            full((1, HF)), full((1, HF)), full((1, HF)), full((1, HF)),
            full((HF, 16)),
            full((E // D, D)),
        ],
        out_specs=[
            pl.BlockSpec((BN, HF), lambda i: (i, 0)),
            pl.BlockSpec((BN, 16), lambda i: (i, 0)),
            pl.BlockSpec((BN, 16), lambda i: (i, 0)),
            full((1, 16)),
        ],
        out_shape=[
            jax.ShapeDtypeStruct((N, HF), jnp.float32),
            jax.ShapeDtypeStruct((N, 16), jnp.float32),
            jax.ShapeDtypeStruct((N, 16), jnp.float32),
            jax.ShapeDtypeStruct((1, 16), jnp.float32),
        ],
        scratch_shapes=[pltpu.VMEM((8, 16), jnp.float32)],
    )(x, wp_t, asrc, atrg, wtp, atp, sel16, ep2d)

    feat, den = _sc_edge(src, trg, ep, tabs, tabt, proj, cvec.reshape(16))

    bias2 = bias.reshape(1, HF)
    bsel = jnp.concatenate(
        [jnp.kron(jnp.eye(H, dtype=jnp.float32), jnp.ones((1, F), jnp.float32)),
         jnp.zeros((8, HF), jnp.float32)], axis=0)
    out = pl.pallas_call(
        _tc2_body,
        grid=(GRID,),
        in_specs=[
            pl.BlockSpec((2, BN, HF), lambda i: (0, i, 0)),
            pl.BlockSpec((2, BN, 16), lambda i: (0, i, 0)),
            pl.BlockSpec((BN, D), lambda i: (i, 0)),
            full((D, HF)),
            full((1, HF)),
            full((16, HF)),
        ],
        out_specs=pl.BlockSpec((BN, HF), lambda i: (i, 0)),
        out_shape=jax.ShapeDtypeStruct((N, HF), jnp.float32),
    )(feat, den, x, ws_t, bias2, bsel)

    return out, edge_index, edge_prob


# speedup vs baseline: 125.7910x; 1.0031x over previous
import functools

import jax
import jax.numpy as jnp
from jax import lax
from jax.experimental import pallas as pl
from jax.experimental.pallas import tpu as pltpu
from jax.experimental.pallas import tpu_sc as plsc

N = 10000
E = 320000
D = 128
H = 8
F = 16
HF = H * F
ROW = 144
C = 80
NWORK = 32
EPT = E // NWORK
NCHUNK = EPT // C
RPT = 624
TAIL = N - 16 * RPT
ZR = 8
BN = 400
GRID = N // BN
EPB = E // D // GRID


def _tc1_body(x_ref, wp_ref, asrc_ref, atrg_ref, wtp_ref, atp_ref,
              sel_ref, ep_ref, proj_ref, tabt_ref,
              cvec_ref, stats_ref):
    pid = pl.program_id(0)

    @pl.when(pid == 0)
    def _init():
        stats_ref[...] = jnp.full((8, 16), -jnp.inf, jnp.float32)
        stats_ref[2:3, :] = jnp.full((1, 16), jnp.max(ep_ref[...]), jnp.float32)

    xb = x_ref[...]
    pb = jnp.dot(xb, wp_ref[...], preferred_element_type=jnp.float32)
    ts = jnp.dot(pb * asrc_ref[...], sel_ref[...],
                 preferred_element_type=jnp.float32)
    tt = jnp.dot(pb * atrg_ref[...], sel_ref[...],
                 preferred_element_type=jnp.float32)
    proj_ref[:, 0:HF] = pb
    proj_ref[:, HF:ROW] = ts
    tabt_ref[...] = tt
    stats_ref[0:1, :] = jnp.maximum(stats_ref[0:1, :],
                                    jnp.max(ts, axis=0, keepdims=True))
    stats_ref[1:2, :] = jnp.maximum(stats_ref[1:2, :],
                                    jnp.max(tt, axis=0, keepdims=True))

    @pl.when(pid == pl.num_programs(0) - 1)
    def _finish():
        c16 = jnp.dot(wtp_ref[...] * atp_ref[...], sel_ref[...],
                      preferred_element_type=jnp.float32)
        bound = jnp.maximum(c16 * stats_ref[2:3, :], 0.0)
        raw = stats_ref[0:1, :] + stats_ref[1:2, :] + bound
        lk = jnp.where(raw > 0.0, raw, 0.2 * raw)
        mp = jnp.max(lk)
        iot = lax.broadcasted_iota(jnp.int32, (1, 16), 1)
        cvec_ref[...] = jnp.where(iot == 15, mp, c16)


def _tc2_body(feat_ref, den_ref, x_ref, ws_ref, bias_ref, bsel_ref, out_ref):
    num = feat_ref[0] + feat_ref[1]
    den = jnp.dot(den_ref[0] + den_ref[1], bsel_ref[...],
                  preferred_element_type=jnp.float32)
    skip = jnp.dot(x_ref[...], ws_ref[...], preferred_element_type=jnp.float32)
    o = num / (den + 1e-16) + skip + bias_ref[...]
    out_ref[...] = jnp.where(o > 0.0, o, jnp.exp(o) - 1.0)


def _sc_edge(pk, tabt, proj, cvec):
    mesh = plsc.VectorSubcoreMesh(core_axis_name="c", subcore_axis_name="s",
                                  num_cores=2, num_subcores=16)

    @functools.partial(
        pl.kernel,
        mesh=mesh,
        compiler_params=pltpu.CompilerParams(use_tc_tiling_on_sc=False,
                                             needs_layout_passes=False),
        out_type=(jax.ShapeDtypeStruct((2, N, HF), jnp.float32),
                  jax.ShapeDtypeStruct((2, N, 16), jnp.float32)),
        scratch_types=[
            [pltpu.VMEM((3, C + 16), jnp.int32)] * 2,
            [pltpu.VMEM((C, 16), jnp.float32)] * 2,
            [pltpu.VMEM((C, ROW), jnp.float32)] * 2,
            pltpu.VMEM((C, ROW), jnp.float32),
            pltpu.VMEM((16,), jnp.float32),
            pltpu.VMEM((ZR, ROW), jnp.float32),
            pltpu.VMEM_SHARED((N, ROW), jnp.float32),
            [pltpu.SemaphoreType.DMA] * 2,
            [pltpu.SemaphoreType.DMA] * 2,
        ],
    )
    def k(pk_h, tabt_h, proj_h, cvec_h,
          feat_h, den_h,
          pk_v, trows, prows, sbuf, cbuf, zbuf,
          accum, isem, gsem):
        cid = lax.axis_index("c")
        sid = lax.axis_index("s")
        wid = sid * 2 + cid

        def issue_idx(k_, p):
            base = wid * EPT + k_ * C
            pltpu.async_copy(pk_h.at[:, pl.ds(base, C)],
                             pk_v[p].at[:, pl.ds(0, C)], isem[p])

        def wait_idx(p):
            pltpu.make_async_copy(pk_h.at[:, pl.ds(0, C)],
                                  pk_v[p].at[:, pl.ds(0, C)], isem[p]).wait()

        def issue_gather(p):
            pltpu.async_copy(proj_h.at[pk_v[p].at[0, pl.ds(0, C)]],
                             prows[p], gsem[p])
            pltpu.async_copy(tabt_h.at[pk_v[p].at[1, pl.ds(0, C)]],
                             trows[p], gsem[p])

        def wait_gather(p):
            pltpu.make_async_copy(proj_h.at[pl.ds(0, C)], prows[p],
                                  gsem[p]).wait()
            pltpu.make_async_copy(tabt_h.at[pl.ds(0, C)], trows[p],
                                  gsem[p]).wait()

        def zrow(r, carry):
            for j in range(ROW // 16):
                zbuf[r, pl.ds(16 * j, 16)] = jnp.zeros((16,), jnp.float32)
            return carry

        issue_idx(0, 0)
        issue_idx(1, 1)
        lax.fori_loop(0, ZR, zrow, 0)

        def zcopy(z, carry):
            pltpu.sync_copy(zbuf, accum.at[pl.ds(sid * RPT + z * ZR, ZR)])
            return carry

        lax.fori_loop(0, RPT // ZR, zcopy, 0)

        @pl.when(sid == 15)
        def _zero_tail():
            pltpu.sync_copy(zbuf.at[pl.ds(0, TAIL)],
                            accum.at[pl.ds(16 * RPT, TAIL)])
        pltpu.sync_copy(cvec_h, cbuf)
        wait_idx(0)
        issue_gather(0)
        plsc.subcore_barrier()

        cv = cbuf[...]
        mp = cv[15]
        lane_lt8 = lax.iota(jnp.int32, 16) < 8

        def compute_chunk(p):
            @plsc.parallel_loop(0, C, unroll=8)
            def _edges(i):
                ev = plsc.bitcast(pk_v[p][2, pl.ds(i, 16)], jnp.float32)
                s = prows[p][i, pl.ds(HF, 16)] + trows[p][i, :] + ev[0] * cv
                s = jnp.where(s > 0.0, s, 0.2 * s)
                e = jnp.exp(s - mp)
                e = jnp.where(lane_lt8, e, 0.0)
                sbuf[i, pl.ds(HF, 16)] = e
                for j in range(H):
                    sbuf[i, pl.ds(16 * j, 16)] = (
                        prows[p][i, pl.ds(16 * j, 16)] * e[j])

            pltpu.sync_copy(sbuf, accum.at[pk_v[p].at[1, pl.ds(0, C)]],
                            add=True)

        def pair(gg, carry):
            k0 = 2 * gg
            wait_gather(0)
            wait_idx(1)
            issue_gather(1)
            compute_chunk(0)
            issue_idx(k0 + 2, 0)
            wait_gather(1)
            wait_idx(0)
            issue_gather(0)
            compute_chunk(1)

            @pl.when(gg < (NCHUNK - 1) // 2 - 1)
            def _more():
                issue_idx(k0 + 3, 1)

            return carry

        lax.fori_loop(0, (NCHUNK - 1) // 2, pair, 0)
        wait_gather(0)
        compute_chunk(0)
        plsc.subcore_barrier()
        pltpu.sync_copy(accum.at[pl.ds(sid * RPT, RPT), pl.ds(0, HF)],
                        feat_h.at[cid, pl.ds(sid * RPT, RPT)])
        pltpu.sync_copy(accum.at[pl.ds(sid * RPT, RPT), pl.ds(HF, 16)],
                        den_h.at[cid, pl.ds(sid * RPT, RPT)])

        @pl.when(sid == 15)
        def _copy_tail():
            pltpu.sync_copy(accum.at[pl.ds(16 * RPT, TAIL), pl.ds(0, HF)],
                            feat_h.at[cid, pl.ds(16 * RPT, TAIL)])
            pltpu.sync_copy(accum.at[pl.ds(16 * RPT, TAIL), pl.ds(HF, 16)],
                            den_h.at[cid, pl.ds(16 * RPT, TAIL)])

    return k(pk, tabt, proj, cvec)


def kernel(x, edge_index, edge_prob, W_proj, W_tp, a_src, a_trg, a_tp,
           W_skip, bias):
    ep = edge_prob.reshape(E)
    pk = jnp.concatenate(
        [edge_index,
         lax.bitcast_convert_type(ep, jnp.int32).reshape(1, E)], axis=0)
    wp_t = W_proj.T
    ws_t = W_skip.T
    asrc = a_src.reshape(1, HF)
    atrg = a_trg.reshape(1, HF)
    atp = a_tp.reshape(1, HF)
    wtp = W_tp.reshape(1, HF)
    sel16 = jnp.concatenate(
        [jnp.kron(jnp.eye(H, dtype=jnp.float32), jnp.ones((F, 1), jnp.float32)),
         jnp.zeros((HF, 8), jnp.float32)], axis=1)
    ep2d = ep.reshape(E // D, D)

    full = lambda shape: pl.BlockSpec(shape, lambda i: (0,) * len(shape))
    proj, tabt, cvec = pl.pallas_call(
        _tc1_body,
        grid=(GRID,),
        in_specs=[
            pl.BlockSpec((BN, D), lambda i: (i, 0)),
            full((D, HF)),
            full((1, HF)), full((1, HF)), full((1, HF)), full((1, HF)),
            full((HF, 16)),
            full((E // D, D)),
        ],
        out_specs=[
            pl.BlockSpec((BN, ROW), lambda i: (i, 0)),
            pl.BlockSpec((BN, 16), lambda i: (i, 0)),
            full((1, 16)),
        ],
        out_shape=[
            jax.ShapeDtypeStruct((N, ROW), jnp.float32),
            jax.ShapeDtypeStruct((N, 16), jnp.float32),
            jax.ShapeDtypeStruct((1, 16), jnp.float32),
        ],
        scratch_shapes=[pltpu.VMEM((8, 16), jnp.float32)],
    )(x, wp_t, asrc, atrg, wtp, atp, sel16, ep2d)

    feat, den = _sc_edge(pk, tabt, proj, cvec.reshape(16))

    bias2 = bias.reshape(1, HF)
    bsel = jnp.concatenate(
        [jnp.kron(jnp.eye(H, dtype=jnp.float32), jnp.ones((1, F), jnp.float32)),
         jnp.zeros((8, HF), jnp.float32)], axis=0)
    out = pl.pallas_call(
        _tc2_body,
        grid=(GRID,),
        in_specs=[
            pl.BlockSpec((2, BN, HF), lambda i: (0, i, 0)),
            pl.BlockSpec((2, BN, 16), lambda i: (0, i, 0)),
            pl.BlockSpec((BN, D), lambda i: (i, 0)),
            full((D, HF)),
            full((1, HF)),
            full((16, HF)),
        ],
        out_specs=pl.BlockSpec((BN, HF), lambda i: (i, 0)),
        out_shape=jax.ShapeDtypeStruct((N, HF), jnp.float32),
    )(feat, den, x, ws_t, bias2, bsel)

    return out, edge_index, edge_prob
